# baseline (speedup 1.0000x reference)
import jax
import jax.numpy as jnp
from jax.experimental import pallas as pl


def _slice_copy(pe_ref, o_ref):
    o_ref[...] = pe_ref[...]


def kernel(x, pe):
    T = x.shape[1]
    D = pe.shape[2]
    return pl.pallas_call(
        _slice_copy,
        grid=(1,),
        out_shape=jax.ShapeDtypeStruct((1, T, D), pe.dtype),
        in_specs=[pl.BlockSpec((1, T, D), lambda i: (0, 0, 0))],
        out_specs=pl.BlockSpec((1, T, D), lambda i: (0, 0, 0)),
    )(pe)
